# baseline (speedup 1.0000x reference)
import jax
import jax.numpy as jnp
from jax import lax
from jax.experimental import pallas as pl
from jax.experimental.pallas import tpu as pltpu
from jax.experimental.pallas import tpu_sc as plsc

NUM_EMB = 1_000_000
D = 32
R = 16
SCALING = 2.0

BLKJ = 2048
NGRID = (NUM_EMB // 4 + BLKJ - 1) // BLKJ
NROW_J = NGRID * BLKJ

NC = 2
NS = 16
NW = NC * NS

LSEQ = 50
FL = 5
NFL = LSEQ // FL


def _fuse_body(wt_ref, a_ref, rhs_ref, o_ref):
    a = a_ref[...]
    wt = wt_ref[...]
    aug = jnp.concatenate([a, wt], axis=0)
    rhs = rhs_ref[...]
    d128 = lax.dot_general(aug.astype(jnp.bfloat16), rhs.astype(jnp.bfloat16),
                           (((0,), (1,)), ((), ())),
                           preferred_element_type=jnp.float32)
    lanes = lax.broadcasted_iota(jnp.int32, (BLKJ, 4 * D), 1)
    r = [d128[BLKJ * c:BLKJ * (c + 1), :] for c in range(4)]
    o_ref[...] = jnp.where(
        lanes < D, r[0],
        jnp.where(lanes < 2 * D, r[1],
                  jnp.where(lanes < 3 * D, r[2], r[3]))).astype(jnp.bfloat16)


def _build_fused(wt, lora_a, rhs):
    return pl.pallas_call(
        _fuse_body,
        grid=(NGRID,),
        in_specs=[
            pl.BlockSpec((D, 4 * BLKJ), lambda i: (0, i)),
            pl.BlockSpec((R, 4 * BLKJ), lambda i: (0, i)),
            pl.BlockSpec((4 * D, R + D), lambda i: (0, 0)),
        ],
        out_specs=pl.BlockSpec((BLKJ, 4 * D), lambda i: (i, 0)),
        out_shape=jax.ShapeDtypeStruct((NROW_J, 4 * D), jnp.bfloat16),
    )(wt, lora_a, rhs)


def _lane_rot(x, iota16, r):
    idx = (iota16 - r) & 15
    dn = lax.GatherDimensionNumbers(
        offset_dims=(), collapsed_slice_dims=(0,), start_index_map=(0,))
    return lax.gather(x, idx[:, None], dn, (1,),
                      mode=lax.GatherScatterMode.PROMISE_IN_BOUNDS)


def _gather_body(fused_hbm, idx_hbm, out_hbm,
                 idx_v, buf_a, buf_b, t_a, t_b,
                 sem_ga, sem_gb, sem_oa, sem_ob):
    wid = lax.axis_index("s") * NC + lax.axis_index("c")
    iota16 = lax.iota(jnp.int32, 16)
    masks = {k: (iota16 & k) == 0 for k in (1, 2, 4, 8)}
    pltpu.sync_copy(idx_hbm.at[:, pl.ds(wid * 128, 128)], idx_v)

    def fire(m, buf, sem):
        for k in range(FL):
            pltpu.async_copy(
                fused_hbm.at[idx_v.at[m * FL + k]],
                buf.at[pl.ds(k * 128, 128)], sem)

    def drain_gathers(m, buf, sem):
        for k in range(FL):
            pltpu.make_async_copy(
                fused_hbm.at[idx_v.at[m * FL + k]],
                buf.at[pl.ds(k * 128, 128)], sem).wait()

    def drain_out(t, sem):
        for k in range(FL):
            for td in range(4):
                pltpu.make_async_copy(
                    out_hbm.at[0, 0, 0], t.at[k, td], sem).wait()

    def transpose_flush(buf, t):
        def blk(n, carry):
            k = n >> 3
            b0 = (n & 7) * 16
            row0 = k * 128 + b0
            v = [buf[row0 + q, pl.ds(0, 16)] for q in range(16)]
            for s in (1, 2, 4, 8):
                nv = list(v)
                for i in range(16):
                    if i & s:
                        continue
                    j = i | s
                    a, b = v[i], v[j]
                    nv[i] = jnp.where(masks[s], a, _lane_rot(b, iota16, s))
                    nv[j] = jnp.where(masks[s], _lane_rot(a, iota16, -s), b)
                v = nv
            for q in range(16):
                lo = lax.bitcast_convert_type(v[q] << 16, jnp.float32)
                hi = lax.bitcast_convert_type(
                    v[q] & jnp.int32(-65536), jnp.float32)
                d0, d1 = 2 * q, 2 * q + 1
                t[k, d0 >> 3, d0 & 7, pl.ds(b0, 16)] = lo
                t[k, d1 >> 3, d1 & 7, pl.ds(b0, 16)] = hi
            return carry
        lax.fori_loop(0, FL * 8, blk, 0)

    def write_out(m, t, sem):
        def kbody(k, carry):
            l = m * FL + k
            for td in range(4):
                pltpu.async_copy(t.at[k, td], out_hbm.at[l, td, wid], sem)
            return carry
        lax.fori_loop(0, FL, kbody, 0)

    fire(0, buf_a, sem_ga)

    def body(i, carry):
        m0 = 2 * i
        fire(m0 + 1, buf_b, sem_gb)
        drain_gathers(m0, buf_a, sem_ga)

        @pl.when(i > 0)
        def _():
            drain_out(t_a, sem_oa)
        transpose_flush(buf_a, t_a)
        write_out(m0, t_a, sem_oa)

        @pl.when(i < (NFL // 2 - 1))
        def _():
            fire(m0 + 2, buf_a, sem_ga)
        drain_gathers(m0 + 1, buf_b, sem_gb)

        @pl.when(i > 0)
        def _():
            drain_out(t_b, sem_ob)
        transpose_flush(buf_b, t_b)
        write_out(m0 + 1, t_b, sem_ob)
        return carry

    lax.fori_loop(0, NFL // 2, body, 0)
    drain_out(t_a, sem_oa)
    drain_out(t_b, sem_ob)


def _sc_gather(fused, idx_t):
    mesh = plsc.VectorSubcoreMesh(core_axis_name="c", subcore_axis_name="s")
    kfn = pl.kernel(
        _gather_body,
        mesh=mesh,
        compiler_params=pltpu.CompilerParams(use_tc_tiling_on_sc=False),
        out_type=jax.ShapeDtypeStruct((LSEQ, 4, NW, 8, 128), jnp.float32),
        scratch_types=[
            pltpu.VMEM((LSEQ, 128), jnp.int32),
            pltpu.VMEM((FL * 128, D // 2), jnp.int32),
            pltpu.VMEM((FL * 128, D // 2), jnp.int32),
            pltpu.VMEM((FL, 4, 8, 128), jnp.float32),
            pltpu.VMEM((FL, 4, 8, 128), jnp.float32),
            pltpu.SemaphoreType.DMA,
            pltpu.SemaphoreType.DMA,
            pltpu.SemaphoreType.DMA,
            pltpu.SemaphoreType.DMA,
        ],
    )
    return kfn(fused, idx_t)


def kernel(x, weight, lora_a, lora_b):
    bsz, lsz = x.shape
    wt = weight.T
    rhs = jnp.tile(jnp.concatenate(
        [lora_b * SCALING, jnp.eye(D, dtype=jnp.float32)], axis=1),
        (4, 1))
    fused128 = _build_fused(wt, lora_a, rhs)
    fused_i32 = lax.bitcast_convert_type(
        fused128.reshape(NROW_J, 2 * D, 2), jnp.int32)
    fused = fused_i32.reshape(NROW_J * 4, D // 2)
    xt = ((x >> 13) << 13) + ((x & 2047) << 2) + ((x >> 11) & 3)
    out5 = _sc_gather(fused, xt.T)
    return out5.transpose(2, 4, 0, 1, 3).reshape(bsz, lsz, D)


# speedup vs baseline: 3.9732x; 3.9732x over previous
import jax
import jax.numpy as jnp
from jax import lax
from jax.experimental import pallas as pl
from jax.experimental.pallas import tpu as pltpu
from jax.experimental.pallas import tpu_sc as plsc

NUM_EMB = 1_000_000
D = 32
R = 16
SCALING = 2.0

BLKJ = 2048
NGRID = (NUM_EMB // 4 + BLKJ - 1) // BLKJ
NROW_J = NGRID * BLKJ

NC = 2
NS = 16
NW = NC * NS

LSEQ = 50
FL = 5
NFL = LSEQ // FL


def _fuse_body(wt_ref, a_ref, rhs_ref, o_ref):
    a = a_ref[...]
    wt = wt_ref[...]
    aug = jnp.concatenate([a, wt], axis=0)
    rhs = rhs_ref[...]
    d128 = lax.dot_general(aug.astype(jnp.bfloat16), rhs.astype(jnp.bfloat16),
                           (((0,), (1,)), ((), ())),
                           preferred_element_type=jnp.float32)
    lanes = lax.broadcasted_iota(jnp.int32, (BLKJ, 4 * D), 1)
    r = [d128[BLKJ * c:BLKJ * (c + 1), :] for c in range(4)]
    o_ref[...] = jnp.where(
        lanes < D, r[0],
        jnp.where(lanes < 2 * D, r[1], jnp.where(lanes < 3 * D, r[2], r[3])))


def _build_fused(wt, lora_a, rhs):
    return pl.pallas_call(
        _fuse_body,
        grid=(NGRID,),
        in_specs=[
            pl.BlockSpec((D, 4 * BLKJ), lambda i: (0, i)),
            pl.BlockSpec((R, 4 * BLKJ), lambda i: (0, i)),
            pl.BlockSpec((4 * D, R + D), lambda i: (0, 0)),
        ],
        out_specs=pl.BlockSpec((BLKJ, 4 * D), lambda i: (i, 0)),
        out_shape=jax.ShapeDtypeStruct((NROW_J, 4 * D), jnp.float32),
    )(wt, lora_a, rhs)


def _lane_rot(x, iota16, r):
    idx = (iota16 - r) & 15
    dn = lax.GatherDimensionNumbers(
        offset_dims=(), collapsed_slice_dims=(0,), start_index_map=(0,))
    return lax.gather(x, idx[:, None], dn, (1,),
                      mode=lax.GatherScatterMode.PROMISE_IN_BOUNDS)


def _gather_body(fused_hbm, idx_hbm, out_hbm,
                 idx_v, buf_a, buf_b, t_a, t_b,
                 sem_ga, sem_gb, sem_oa, sem_ob):
    wid = lax.axis_index("s") * NC + lax.axis_index("c")
    iota16 = lax.iota(jnp.int32, 16)
    masks = {k: (iota16 & k) == 0 for k in (1, 2, 4, 8)}
    pltpu.sync_copy(idx_hbm.at[:, pl.ds(wid * 128, 128)], idx_v)

    def fire(m, buf, sem):
        for k in range(FL):
            pltpu.async_copy(
                fused_hbm.at[idx_v.at[m * FL + k]],
                buf.at[pl.ds(k * 128, 128)], sem)

    def drain_gathers(m, buf, sem):
        for k in range(FL):
            pltpu.make_async_copy(
                fused_hbm.at[idx_v.at[m * FL + k]],
                buf.at[pl.ds(k * 128, 128)], sem).wait()

    def drain_out(t, sem):
        for k in range(FL):
            for td in range(4):
                pltpu.make_async_copy(
                    out_hbm.at[0, 0, 0], t.at[k, td], sem).wait()

    def transpose_flush(buf, t):
        def blk(n, carry):
            k = n >> 4
            col0 = ((n >> 3) & 1) * 16
            b0 = (n & 7) * 16
            row0 = k * 128 + b0
            v = [buf[row0 + q, pl.ds(col0, 16)] for q in range(16)]
            for s in (1, 2, 4, 8):
                nv = list(v)
                for i in range(16):
                    if i & s:
                        continue
                    j = i | s
                    a, b = v[i], v[j]
                    nv[i] = jnp.where(masks[s], a, _lane_rot(b, iota16, s))
                    nv[j] = jnp.where(masks[s], _lane_rot(a, iota16, -s), b)
                v = nv
            for q in range(16):
                d = col0 + q
                t[k, d >> 3, d & 7, pl.ds(b0, 16)] = v[q]
            return carry
        lax.fori_loop(0, FL * 16, blk, 0)

    def write_out(m, t, sem):
        def kbody(k, carry):
            l = m * FL + k
            for td in range(4):
                pltpu.async_copy(t.at[k, td], out_hbm.at[l, td, wid], sem)
            return carry
        lax.fori_loop(0, FL, kbody, 0)

    fire(0, buf_a, sem_ga)

    def body(i, carry):
        m0 = 2 * i
        fire(m0 + 1, buf_b, sem_gb)
        drain_gathers(m0, buf_a, sem_ga)

        @pl.when(i > 0)
        def _():
            drain_out(t_a, sem_oa)
        transpose_flush(buf_a, t_a)
        write_out(m0, t_a, sem_oa)

        @pl.when(i < (NFL // 2 - 1))
        def _():
            fire(m0 + 2, buf_a, sem_ga)
        drain_gathers(m0 + 1, buf_b, sem_gb)

        @pl.when(i > 0)
        def _():
            drain_out(t_b, sem_ob)
        transpose_flush(buf_b, t_b)
        write_out(m0 + 1, t_b, sem_ob)
        return carry

    lax.fori_loop(0, NFL // 2, body, 0)
    drain_out(t_a, sem_oa)
    drain_out(t_b, sem_ob)


def _sc_gather(fused, idx_t):
    mesh = plsc.VectorSubcoreMesh(core_axis_name="c", subcore_axis_name="s")
    kfn = pl.kernel(
        _gather_body,
        mesh=mesh,
        compiler_params=pltpu.CompilerParams(use_tc_tiling_on_sc=False),
        out_type=jax.ShapeDtypeStruct((LSEQ, 4, NW, 8, 128), jnp.float32),
        scratch_types=[
            pltpu.VMEM((LSEQ, 128), jnp.int32),
            pltpu.VMEM((FL * 128, D), jnp.float32),
            pltpu.VMEM((FL * 128, D), jnp.float32),
            pltpu.VMEM((FL, 4, 8, 128), jnp.float32),
            pltpu.VMEM((FL, 4, 8, 128), jnp.float32),
            pltpu.SemaphoreType.DMA,
            pltpu.SemaphoreType.DMA,
            pltpu.SemaphoreType.DMA,
            pltpu.SemaphoreType.DMA,
        ],
    )
    return kfn(fused, idx_t)


def kernel(x, weight, lora_a, lora_b):
    bsz, lsz = x.shape
    wt = weight.T
    rhs = jnp.tile(jnp.concatenate(
        [lora_b * SCALING, jnp.eye(D, dtype=jnp.float32)], axis=1),
        (4, 1))
    fused128 = _build_fused(wt, lora_a, rhs)
    fused = fused128.reshape(NROW_J * 4, D)
    xt = ((x >> 13) << 13) + ((x & 2047) << 2) + ((x >> 11) & 3)
    out5 = _sc_gather(fused, xt.T)
    return out5.transpose(2, 4, 0, 1, 3).reshape(bsz, lsz, D)
